# baseline (speedup 1.0000x reference)
import functools
import math

import jax
import jax.numpy as jnp
from jax import lax
from jax.experimental import pallas as pl
from jax.experimental.pallas import tpu as pltpu

_PADL = 64


def _fused_kernel(x_ref, w1_ref, b1_ref, w2_ref, b2_ref, ma_ref, pp_ref, o_ref,
                  yp_ref, *, h, w, c, c_mid, k_enc, k_up, scale, eg):
    xc = x_ref[0]
    y = lax.dot_general(xc, w1_ref[...], (((0,), (0,)), ((), ())),
                        preferred_element_type=jnp.float32) + b1_ref[...]
    y = y * (1.0 / (1.0 + jnp.exp(-y)))

    pe = k_enc // 2
    yp_ref[...] = jnp.zeros_like(yp_ref)
    yp_ref[pe:pe + h, pe:pe + w, :] = y.reshape(h, w, c_mid)
    yp = yp_ref[...]
    etaps = [yp[di:di + h, dj:dj + w, :]
             for di in range(k_enc) for dj in range(k_enc)]
    patches = jnp.concatenate(etaps, axis=-1).reshape(h * w,
                                                      k_enc * k_enc * c_mid)
    z = lax.dot_general(w2_ref[...], patches, (((0,), (1,)), ((), ())),
                        preferred_element_type=jnp.float32) + b2_ref[...]

    ss = scale * scale
    zg = z.reshape(ss, eg, h * w)
    mx = jnp.max(zg, axis=1, keepdims=True)
    ex = jnp.exp(zg - mx)
    sm = jnp.sum(ex, axis=1, keepdims=True)
    p = (ex * pl.reciprocal(sm, approx=True)).reshape(ss * eg, h * w)
    pm = p * ma_ref[...]

    zpad = jnp.zeros((c, _PADL), jnp.float32)
    xcp = jnp.concatenate([zpad, xc, zpad], axis=1)
    pu = k_up // 2
    xtaps = []
    for di in range(k_up):
        for dj in range(k_up):
            off = _PADL + (di - pu) * w + (dj - pu)
            xtaps.append(xcp[:, off:off + h * w])

    accs = []
    for q in range(ss):
        acc = pm[q * eg:q * eg + 1, :] * xtaps[0]
        for k in range(1, k_up * k_up):
            acc = acc + pm[q * eg + k:q * eg + k + 1, :] * xtaps[k]
        accs.append(acc)

    qb = jnp.concatenate(accs, axis=1)
    o_ref[0] = jnp.dot(qb, pp_ref[...],
                       preferred_element_type=jnp.float32)


def kernel(x, w1, w2, bn1_gamma, bn1_beta, bn1_mean, bn1_var,
           bn2_gamma, bn2_beta, bn2_mean, bn2_var):
    b, c, h, w = x.shape
    c_mid = w1.shape[0]
    c_enc, k_enc = w2.shape[0], w2.shape[2]
    k_up, scale = 5, 2
    ss = scale * scale
    c_pad = 128
    eg = c_pad // ss
    h_, w_ = h * scale, w * scale
    eps = 1e-5

    x = x.astype(jnp.float32)

    s1 = bn1_gamma / jnp.sqrt(bn1_var + eps)
    b1 = bn1_beta - bn1_mean * s1
    w1f = w1.reshape(c_mid, c).T * s1[None, :]

    s2 = bn2_gamma / jnp.sqrt(bn2_var + eps)
    b2 = bn2_beta - bn2_mean * s2
    w2f = jnp.transpose(w2 * s2[:, None, None, None], (2, 3, 1, 0))
    w2f = w2f.reshape(k_enc * k_enc * c_mid, c_enc)

    e_ar = jnp.arange(c_enc)
    newcol = (e_ar % ss) * eg + e_ar // ss
    w2p = jnp.zeros((k_enc * k_enc * c_mid, c_pad), jnp.float32)
    w2p = w2p.at[:, newcol].set(w2f)
    b2p = jnp.full((c_pad,), -1e30, jnp.float32).at[newcol].set(b2)

    pu = k_up // 2
    r_ar = jnp.arange(c_pad)
    dj_r = (r_ar % eg) % k_up
    xg = jnp.arange(h * w) % w
    ma = ((xg[None, :] >= pu - dj_r[:, None])
          & (xg[None, :] <= w - 1 + pu - dj_r[:, None])).astype(jnp.float32)

    m_ar = jnp.arange(ss * h * w)
    q_m, r_m = m_ar // (h * w), m_ar % (h * w)
    oy_m = (r_m // w) * scale + q_m // scale
    ox_m = (r_m % w) * scale + q_m % scale
    pp = jnp.zeros((ss * h * w, ss * h * w), jnp.float32)
    pp = pp.at[m_ar, oy_m * (w * scale) + ox_m].set(1.0)

    out_flat = pl.pallas_call(
        functools.partial(_fused_kernel, h=h, w=w, c=c, c_mid=c_mid,
                          k_enc=k_enc, k_up=k_up, scale=scale, eg=eg),
        out_shape=jax.ShapeDtypeStruct((b, c, h_ * w_), jnp.float32),
        grid=(b,),
        in_specs=[pl.BlockSpec((1, c, h * w), lambda i: (i, 0, 0)),
                  pl.BlockSpec((c, c_mid), lambda i: (0, 0)),
                  pl.BlockSpec((1, c_mid), lambda i: (0, 0)),
                  pl.BlockSpec((k_enc * k_enc * c_mid, c_pad),
                               lambda i: (0, 0)),
                  pl.BlockSpec((c_pad, 1), lambda i: (0, 0)),
                  pl.BlockSpec((c_pad, h * w), lambda i: (0, 0)),
                  pl.BlockSpec((ss * h * w, ss * h * w), lambda i: (0, 0))],
        out_specs=pl.BlockSpec((1, c, h_ * w_), lambda i: (i, 0, 0)),
        scratch_shapes=[pltpu.VMEM((h + 2 * (k_enc // 2), w + 2 * (k_enc // 2),
                                    c_mid), jnp.float32)],
        compiler_params=pltpu.CompilerParams(
            dimension_semantics=("parallel",),
            vmem_limit_bytes=32 * 1024 * 1024),
    )(x.reshape(b, c, h * w), w1f, b1[None, :], w2p, b2p[:, None], ma, pp)

    return out_flat.reshape(b, c, h_, w_)


# speedup vs baseline: 1.0254x; 1.0254x over previous
import functools
import math

import jax
import jax.numpy as jnp
from jax import lax
from jax.experimental import pallas as pl
from jax.experimental.pallas import tpu as pltpu

_PADL = 64


def _fused_kernel(x_ref, w1_ref, b1_ref, w2_ref, b2_ref, ma_ref, pp_ref, o_ref,
                  yp_ref, xcp_ref, *, h, w, c, c_mid, k_enc, k_up, scale, eg):
    xc = x_ref[0]
    y = lax.dot_general(xc, w1_ref[...], (((0,), (0,)), ((), ())),
                        preferred_element_type=jnp.float32) + b1_ref[...]
    y = y * (1.0 / (1.0 + jnp.exp(-y)))

    pe = k_enc // 2
    yp_ref[...] = jnp.zeros_like(yp_ref)
    yp_ref[pe:pe + h, pe:pe + w, :] = y.reshape(h, w, c_mid)
    yp = yp_ref[...]
    etaps = [yp[di:di + h, dj:dj + w, :]
             for di in range(k_enc) for dj in range(k_enc)]
    patches = jnp.concatenate(etaps, axis=-1).reshape(h * w,
                                                      k_enc * k_enc * c_mid)
    z = lax.dot_general(w2_ref[...], patches, (((0,), (1,)), ((), ())),
                        preferred_element_type=jnp.float32) + b2_ref[...]

    ss = scale * scale
    zg = z.reshape(ss, eg, h * w)
    mx = jnp.max(zg, axis=1, keepdims=True)
    ex = jnp.exp(zg - mx)
    sm = jnp.sum(ex, axis=1, keepdims=True)
    p = (ex * pl.reciprocal(sm, approx=True)).reshape(ss * eg, h * w)
    pm = p * ma_ref[...]

    xcp_ref[:, :_PADL] = jnp.zeros((c, _PADL), jnp.float32)
    xcp_ref[:, _PADL + h * w:] = jnp.zeros((c, _PADL), jnp.float32)
    xcp_ref[:, _PADL:_PADL + h * w] = xc
    pu = k_up // 2
    accs = [None] * ss
    for k in range(k_up * k_up):
        di, dj = k // k_up, k % k_up
        off = _PADL + (di - pu) * w + (dj - pu)
        tap = xcp_ref[:, off:off + h * w]
        for q in range(ss):
            t = pm[q * eg + k:q * eg + k + 1, :] * tap
            accs[q] = t if k == 0 else accs[q] + t

    qb = jnp.concatenate(accs, axis=1)
    o_ref[0] = jnp.dot(qb.astype(jnp.bfloat16), pp_ref[...],
                       preferred_element_type=jnp.float32)


def kernel(x, w1, w2, bn1_gamma, bn1_beta, bn1_mean, bn1_var,
           bn2_gamma, bn2_beta, bn2_mean, bn2_var):
    b, c, h, w = x.shape
    c_mid = w1.shape[0]
    c_enc, k_enc = w2.shape[0], w2.shape[2]
    k_up, scale = 5, 2
    ss = scale * scale
    c_pad = 128
    eg = c_pad // ss
    h_, w_ = h * scale, w * scale
    eps = 1e-5

    x = x.astype(jnp.float32)

    s1 = bn1_gamma / jnp.sqrt(bn1_var + eps)
    b1 = bn1_beta - bn1_mean * s1
    w1f = w1.reshape(c_mid, c).T * s1[None, :]

    s2 = bn2_gamma / jnp.sqrt(bn2_var + eps)
    b2 = bn2_beta - bn2_mean * s2
    w2f = jnp.transpose(w2 * s2[:, None, None, None], (2, 3, 1, 0))
    w2f = w2f.reshape(k_enc * k_enc * c_mid, c_enc)

    cn = jnp.arange(c_pad)
    k_c, q_c = cn % eg, cn // eg
    valid = k_c < k_up * k_up
    e_src = jnp.clip(k_c * ss + q_c, 0, c_enc - 1)
    w2p = jnp.where(valid[None, :], w2f[:, e_src], 0.0)
    b2p = jnp.where(valid, b2[e_src], -1e30)

    pu = k_up // 2
    r_ar = jnp.arange(c_pad)
    dj_r = (r_ar % eg) % k_up
    xg = jnp.arange(h * w) % w
    ma = ((xg[None, :] >= pu - dj_r[:, None])
          & (xg[None, :] <= w - 1 + pu - dj_r[:, None])).astype(jnp.float32)

    m_ar = jnp.arange(ss * h * w)
    q_m, r_m = m_ar // (h * w), m_ar % (h * w)
    oy_m = (r_m // w) * scale + q_m // scale
    ox_m = (r_m % w) * scale + q_m % scale
    pp = (m_ar[None, :] == (oy_m * (w * scale) + ox_m)[:, None]
          ).astype(jnp.bfloat16)

    out_flat = pl.pallas_call(
        functools.partial(_fused_kernel, h=h, w=w, c=c, c_mid=c_mid,
                          k_enc=k_enc, k_up=k_up, scale=scale, eg=eg),
        out_shape=jax.ShapeDtypeStruct((b, c, h_ * w_), jnp.float32),
        grid=(b,),
        in_specs=[pl.BlockSpec((1, c, h * w), lambda i: (i, 0, 0)),
                  pl.BlockSpec((c, c_mid), lambda i: (0, 0)),
                  pl.BlockSpec((1, c_mid), lambda i: (0, 0)),
                  pl.BlockSpec((k_enc * k_enc * c_mid, c_pad),
                               lambda i: (0, 0)),
                  pl.BlockSpec((c_pad, 1), lambda i: (0, 0)),
                  pl.BlockSpec((c_pad, h * w), lambda i: (0, 0)),
                  pl.BlockSpec((ss * h * w, ss * h * w), lambda i: (0, 0))],
        out_specs=pl.BlockSpec((1, c, h_ * w_), lambda i: (i, 0, 0)),
        scratch_shapes=[pltpu.VMEM((h + 2 * (k_enc // 2), w + 2 * (k_enc // 2),
                                    c_mid), jnp.float32),
                        pltpu.VMEM((c, h * w + 2 * _PADL), jnp.float32)],
        compiler_params=pltpu.CompilerParams(
            dimension_semantics=("parallel",),
            vmem_limit_bytes=32 * 1024 * 1024),
    )(x.reshape(b, c, h * w), w1f, b1[None, :], w2p, b2p[:, None], ma, pp)

    return out_flat.reshape(b, c, h_, w_)
